# Initial kernel scaffold; baseline (speedup 1.0000x reference)
import functools

import jax
import jax.numpy as jnp
from jax import lax
from jax.experimental import pallas as pl
from jax.experimental.pallas import tpu as pltpu
from jax.experimental.pallas import tpu_sc as plsc

N = 10000
E = 320000
D = 128
H = 128
C = 64

NC = 2
NS = 16
NW = NC * NS


def _make_segsum(n, e, w, k):
    epw = e // NW
    assert e % NW == 0
    niter = epw // k
    assert epw % k == 0 and k % 8 == 0
    rows = n // NS
    assert n % NS == 0

    mesh = plsc.VectorSubcoreMesh(core_axis_name="c", subcore_axis_name="s")

    @functools.partial(
        pl.kernel,
        out_type=jax.ShapeDtypeStruct((NC, n, w), jnp.float32),
        mesh=mesh,
        scratch_types=[
            pltpu.VMEM((k,), jnp.int32),
            pltpu.VMEM((k,), jnp.int32),
            pltpu.VMEM((k, w), jnp.float32),
            pltpu.VMEM_SHARED((n, w), jnp.float32),
            pltpu.SemaphoreType.DMA,
        ],
    )
    def segsum(h_hbm, src_hbm, dst_hbm, zeros_hbm, out_hbm,
               src_v, dst_v, rows_v, acc, sem):
        c = lax.axis_index("c")
        s = lax.axis_index("s")
        row0 = s * rows
        pltpu.sync_copy(zeros_hbm.at[pl.ds(row0, rows)],
                        acc.at[pl.ds(row0, rows)])
        plsc.subcore_barrier()

        ebase = c * (e // NC) + s * epw

        def body(i, carry):
            b = ebase + i * k
            pltpu.sync_copy(src_hbm.at[pl.ds(b, k)], src_v)
            pltpu.sync_copy(dst_hbm.at[pl.ds(b, k)], dst_v)
            pltpu.async_copy(h_hbm.at[src_v], rows_v, sem).wait()
            pltpu.sync_copy(rows_v, acc.at[dst_v], add=True)
            return carry

        lax.fori_loop(0, niter, body, 0)
        plsc.subcore_barrier()
        pltpu.sync_copy(acc.at[pl.ds(row0, rows)],
                        out_hbm.at[c, pl.ds(row0, rows)])

    return segsum


_segsum_128 = _make_segsum(N, E, H, 80)
_segsum_64 = _make_segsum(N, E, C, 80)


def _make_linear(n, din, dout, bn, relu):
    def body(p_ref, wt_ref, b_ref, o_ref):
        x = p_ref[0] + p_ref[1]
        y = jnp.dot(x, wt_ref[...], preferred_element_type=jnp.float32)
        y = y + b_ref[...]
        o_ref[...] = jnp.maximum(y, 0.0) if relu else y

    return pl.pallas_call(
        body,
        grid=(n // bn,),
        in_specs=[
            pl.BlockSpec((2, bn, din), lambda i: (0, i, 0)),
            pl.BlockSpec((din, dout), lambda i: (0, 0)),
            pl.BlockSpec((1, dout), lambda i: (0, 0)),
        ],
        out_specs=pl.BlockSpec((bn, dout), lambda i: (i, 0)),
        out_shape=jax.ShapeDtypeStruct((n, dout), jnp.float32),
    )


def _make_linear_fused2(n, din, dh, dc, bn):
    def body(p_ref, w1t_ref, b1_ref, w2t_ref, h_ref, y_ref):
        x = p_ref[0] + p_ref[1]
        h = jnp.dot(x, w1t_ref[...], preferred_element_type=jnp.float32)
        h = jnp.maximum(h + b1_ref[...], 0.0)
        h_ref[...] = h
        y_ref[...] = jnp.dot(h, w2t_ref[...],
                             preferred_element_type=jnp.float32)

    return pl.pallas_call(
        body,
        grid=(n // bn,),
        in_specs=[
            pl.BlockSpec((2, bn, din), lambda i: (0, i, 0)),
            pl.BlockSpec((din, dh), lambda i: (0, 0)),
            pl.BlockSpec((1, dh), lambda i: (0, 0)),
            pl.BlockSpec((dh, dc), lambda i: (0, 0)),
        ],
        out_specs=[
            pl.BlockSpec((bn, dh), lambda i: (i, 0)),
            pl.BlockSpec((bn, dc), lambda i: (i, 0)),
        ],
        out_shape=[
            jax.ShapeDtypeStruct((n, dh), jnp.float32),
            jax.ShapeDtypeStruct((n, dc), jnp.float32),
        ],
    )


def _make_logsoftmax(n, dc, bn):
    def body(p_ref, b_ref, o_ref):
        z = p_ref[0] + p_ref[1] + b_ref[...]
        m = jnp.max(z, axis=1, keepdims=True)
        ez = z - m
        lse = jnp.log(jnp.sum(jnp.exp(ez), axis=1, keepdims=True))
        o_ref[...] = ez - lse

    return pl.pallas_call(
        body,
        grid=(n // bn,),
        in_specs=[
            pl.BlockSpec((2, bn, dc), lambda i: (0, i, 0)),
            pl.BlockSpec((1, dc), lambda i: (0, 0)),
        ],
        out_specs=pl.BlockSpec((bn, dc), lambda i: (i, 0)),
        out_shape=jax.ShapeDtypeStruct((n, dc), jnp.float32),
    )


_BN = 1000
_linear0 = _make_linear(N, D, H, _BN, True)
_linear1_fused = _make_linear_fused2(N, H, H, C, _BN)
_logsoftmax = _make_logsoftmax(N, C, _BN)


def kernel(features, labels, mask, edge_index, W0, b0, W1, b1, W2, b2):
    src = edge_index[0]
    dst = edge_index[1]
    zeros128 = jnp.zeros((N, H), jnp.float32)
    zeros64 = jnp.zeros((N, C), jnp.float32)
    w0t = W0.T
    w1t = W1.T
    w2t = W2.T
    b0r = b0.reshape(1, H)
    b1r = b1.reshape(1, H)
    b2r = b2.reshape(1, C)

    p0 = _segsum_128(features, src, dst, zeros128)
    h0 = _linear0(p0, w0t, b0r)
    p1 = _segsum_128(h0, src, dst, zeros128)
    h1, y2 = _linear1_fused(p1, w1t, b1r, w2t)
    p2 = _segsum_64(y2, src, dst, zeros64)
    out = _logsoftmax(p2, b2r)
    return out


# speedup vs baseline: 4.5065x; 4.5065x over previous
import functools

import jax
import jax.numpy as jnp
from jax import lax
from jax.experimental import pallas as pl
from jax.experimental.pallas import tpu as pltpu
from jax.experimental.pallas import tpu_sc as plsc

N = 10000
E = 320000
D = 128
H = 128
C = 64

NC = 2
NS = 16
NW = NC * NS
NP = 10112


def _make_segsum(n_in, n_out, w, k):
    epw = E // NW
    assert E % NW == 0
    niter = epw // k
    assert epw % k == 0 and k % 8 == 0
    rows = n_out // NS
    assert n_out % NS == 0 and rows % 8 == 0

    mesh = plsc.VectorSubcoreMesh(core_axis_name="c", subcore_axis_name="s")

    @functools.partial(
        pl.kernel,
        out_type=jax.ShapeDtypeStruct((NC, n_out, w), jnp.float32),
        mesh=mesh,
        scratch_types=[
            pltpu.VMEM((k,), jnp.int32),
            pltpu.VMEM((k,), jnp.int32),
            pltpu.VMEM((k, w), jnp.float32),
            pltpu.VMEM_SHARED((n_out, w), jnp.float32),
            pltpu.SemaphoreType.DMA,
        ],
    )
    def segsum(h_hbm, src_hbm, dst_hbm, zeros_hbm, out_hbm,
               src_v, dst_v, rows_v, acc, sem):
        c = lax.axis_index("c")
        s = lax.axis_index("s")
        row0 = s * rows
        pltpu.sync_copy(zeros_hbm.at[pl.ds(row0, rows)],
                        acc.at[pl.ds(row0, rows)])
        plsc.subcore_barrier()

        ebase = c * (E // NC) + s * epw

        def body(i, carry):
            b = ebase + i * k
            pltpu.sync_copy(src_hbm.at[pl.ds(b, k)], src_v)
            pltpu.sync_copy(dst_hbm.at[pl.ds(b, k)], dst_v)
            pltpu.async_copy(h_hbm.at[src_v], rows_v, sem).wait()
            pltpu.sync_copy(rows_v, acc.at[dst_v], add=True)
            return carry

        lax.fori_loop(0, niter, body, 0)
        plsc.subcore_barrier()
        pltpu.sync_copy(acc.at[pl.ds(row0, rows)],
                        out_hbm.at[c, pl.ds(row0, rows)])

    return segsum


_segsum_feat = _make_segsum(N, NP, H, 80)
_segsum_hid = _make_segsum(NP, NP, H, 80)


def _make_linear(n, din, dout, bn, relu):
    def body(p_ref, wt_ref, b_ref, o_ref):
        x = p_ref[0] + p_ref[1]
        y = jnp.dot(x, wt_ref[...], preferred_element_type=jnp.float32)
        y = y + b_ref[...]
        o_ref[...] = jnp.maximum(y, 0.0) if relu else y

    return pl.pallas_call(
        body,
        grid=(n // bn,),
        in_specs=[
            pl.BlockSpec((2, bn, din), lambda i: (0, i, 0)),
            pl.BlockSpec((din, dout), lambda i: (0, 0)),
            pl.BlockSpec((1, dout), lambda i: (0, 0)),
        ],
        out_specs=pl.BlockSpec((bn, dout), lambda i: (i, 0)),
        out_shape=jax.ShapeDtypeStruct((n, dout), jnp.float32),
    )


def _make_final(n, din, dc, bn):
    def body(p_ref, wt_ref, b_ref, o_ref):
        x = p_ref[0] + p_ref[1]
        z = jnp.dot(x, wt_ref[...], preferred_element_type=jnp.float32)
        z = z + b_ref[...]
        m = jnp.max(z, axis=1, keepdims=True)
        ez = z - m
        lse = jnp.log(jnp.sum(jnp.exp(ez), axis=1, keepdims=True))
        o_ref[...] = ez - lse

    return pl.pallas_call(
        body,
        grid=(n // bn,),
        in_specs=[
            pl.BlockSpec((2, bn, din), lambda i: (0, i, 0)),
            pl.BlockSpec((din, dc), lambda i: (0, 0)),
            pl.BlockSpec((1, dc), lambda i: (0, 0)),
        ],
        out_specs=pl.BlockSpec((bn, dc), lambda i: (i, 0)),
        out_shape=jax.ShapeDtypeStruct((n, dc), jnp.float32),
    )


_BN = 632
_linear0 = _make_linear(NP, D, H, _BN, True)
_linear1 = _make_linear(NP, H, H, _BN, True)
_final = _make_final(NP, H, C, _BN)


def kernel(features, labels, mask, edge_index, W0, b0, W1, b1, W2, b2):
    src = edge_index[0]
    dst = edge_index[1]
    zeros128 = jnp.zeros((NP, H), jnp.float32)
    w0t = W0.T
    w1t = W1.T
    w2t = W2.T
    b0r = b0.reshape(1, H)
    b1r = b1.reshape(1, H)
    b2r = b2.reshape(1, C)

    p0 = _segsum_feat(features, src, dst, zeros128)
    h0 = _linear0(p0, w0t, b0r)
    p1 = _segsum_hid(h0, src, dst, zeros128)
    h1 = _linear1(p1, w1t, b1r)
    p2 = _segsum_hid(h1, src, dst, zeros128)
    out = _final(p2, w2t, b2r)
    return out[:N]
